# baseline (speedup 1.0000x reference)
import jax
import jax.numpy as jnp
from jax.experimental import pallas as pl
from jax.experimental.pallas import tpu as pltpu

S = 2048
HID = 128
BSZ = 4
STACK = BSZ * HID
TIME_STEPS = 10
TEMPORAL_STEPS = 3
NSPLIT = 2
RSPLIT = S // NSPLIT
NSTAGE = 4


def _swish(v):
    return v * jax.nn.sigmoid(v)


def _dot(a, b, dims):
    return jax.lax.dot_general(
        a, b, (dims, ((), ())), preferred_element_type=jnp.float32)


def _blockmix(w, xs):
    return jnp.concatenate(
        [_dot(w, xs[b * HID:(b + 1) * HID], ((0,), (0,))) for b in range(BSZ)],
        axis=0)


def _col(vec_ref):
    return jnp.transpose(jnp.reshape(vec_ref[...], (1, -1)))


def _col4(vec_ref):
    c = _col(vec_ref)
    return jnp.concatenate([c] * BSZ, axis=0)


def _fused_kernel(x0t_ref, b1_hbm, b2_hbm,
                  w_enc0_ref, b_enc0_ref, w2f_ref, b_enc2_ref,
                  w_c0_ref, w_c2_ref, alpha_ref,
                  w_f_ref, b_f_ref,
                  out_ref, stage, b2bf_s, sems):
    def _issue(i):
        src = b1_hbm if i < NSPLIT else b2_hbm
        c = i % NSPLIT
        pltpu.make_async_copy(src.at[pl.ds(c * RSPLIT, RSPLIT)],
                              stage.at[i % NSTAGE],
                              sems.at[i]).start()

    def _wait(i):
        src = b1_hbm if i < NSPLIT else b2_hbm
        c = i % NSPLIT
        pltpu.make_async_copy(src.at[pl.ds(c * RSPLIT, RSPLIT)],
                              stage.at[i % NSTAGE],
                              sems.at[i]).wait()

    for i in range(NSTAGE):
        _issue(i)

    alpha = alpha_ref[0]
    w_enc0 = w_enc0_ref[...]
    x0h = _swish(jnp.concatenate(
        [_dot(w_enc0, x0t_ref[:, b * S:(b + 1) * S], ((0,), (0,)))
         for b in range(BSZ)], axis=0) + _col4(b_enc0_ref))

    x0h_bf = x0h.astype(jnp.bfloat16)
    t1 = None
    for i in range(NSPLIT):
        _wait(i)
        p = _dot(x0h_bf[:, i * RSPLIT:(i + 1) * RSPLIT],
                 stage[i % NSTAGE].astype(jnp.bfloat16), ((1,), (0,)))
        t1 = p if t1 is None else t1 + p

    w_f = w_f_ref[...]
    b_f = _col(b_f_ref)
    l0_1 = _blockmix(w_c0_ref[...], x0h)
    acc = [_dot(w_f[0:HID], x0h[b * HID:(b + 1) * HID], ((0,), (0,)))
           for b in range(BSZ)]
    t1_bf = t1.astype(jnp.bfloat16)

    t2 = None
    for i in range(NSPLIT, 2 * NSPLIT):
        c = i % NSPLIT
        _wait(i)
        half = stage[i % NSTAGE].astype(jnp.bfloat16)
        b2bf_s[c * RSPLIT:(c + 1) * RSPLIT, :] = half
        p = _dot(t1_bf[:, c * RSPLIT:(c + 1) * RSPLIT], half, ((1,), (0,)))
        t2 = p if t2 is None else t2 + p
    x2h = _blockmix(w2f_ref[...], t2) + _col4(b_enc2_ref)

    B2 = b2bf_s[...]

    x2h_bf = x2h.astype(jnp.bfloat16)
    up0 = _dot(x2h_bf, B2, ((1,), (0,)))
    x0h_1 = _swish(up0 + alpha * (l0_1 - up0))
    x2h_1 = _swish(_blockmix(w_c2_ref[...], x2h))
    acc = [a + _dot(w_f[HID:2 * HID], x0h_1[b * HID:(b + 1) * HID],
                    ((0,), (0,))) for b, a in enumerate(acc)]

    x2h1_bf = x2h_1.astype(jnp.bfloat16)
    l0_2 = _blockmix(w_c0_ref[...], x0h_1)
    up0b = _dot(x2h1_bf, B2, ((1,), (0,)))
    x0h_2 = _swish(up0b + alpha * (l0_2 - up0b))

    for b in range(BSZ):
        a = acc[b] + _dot(w_f[2 * HID:3 * HID],
                          x0h_2[b * HID:(b + 1) * HID], ((0,), (0,)))
        out_ref[b] = _swish(a + b_f)


def kernel(x, pos, batch, triangles, B1, B2, W_enc0, b_enc0, theta_edge,
           theta_tri, W_enc1, b_enc1, W_enc2, b_enc2, W_conv0, W_conv1,
           W_conv2, alpha, W_tproj, b_tproj, W_dec, b_dec):
    vfull = lambda shp: pl.BlockSpec(shp, lambda: (0,) * len(shp))
    hbm = pl.BlockSpec(memory_space=pl.ANY)
    smem1 = pl.BlockSpec(memory_space=pltpu.SMEM)

    x0t = jnp.concatenate([x.T, pos.T], axis=0)
    hp = 'highest'
    w2f = jnp.dot(jnp.dot(theta_edge, theta_tri, precision=hp), W_enc2,
                  precision=hp)
    w_f = jnp.dot(W_tproj, W_dec, precision=hp)
    b_f = jnp.dot(b_tproj, W_dec, precision=hp) + b_dec

    out = pl.pallas_call(
        _fused_kernel,
        in_specs=[
            vfull((5, BSZ * S)),
            hbm, hbm,
            vfull((5, HID)), vfull((HID,)),
            vfull((HID, HID)), vfull((HID,)),
            vfull((HID, HID)), vfull((HID, HID)),
            smem1,
            vfull((HID * TEMPORAL_STEPS, TIME_STEPS)), vfull((TIME_STEPS,)),
        ],
        out_specs=vfull((BSZ, TIME_STEPS, S)),
        out_shape=jax.ShapeDtypeStruct((BSZ, TIME_STEPS, S), jnp.float32),
        scratch_shapes=[
            pltpu.VMEM((NSTAGE, RSPLIT, S), jnp.float32),
            pltpu.VMEM((S, S), jnp.bfloat16),
            pltpu.SemaphoreType.DMA((2 * NSPLIT,)),
        ],
        compiler_params=pltpu.CompilerParams(
            vmem_limit_bytes=110 * 1024 * 1024),
    )(
        x0t, B1, B2,
        W_enc0, b_enc0, w2f, b_enc2,
        W_conv0, W_conv2, alpha.reshape(1),
        w_f, b_f,
    )
    return out


# speedup vs baseline: 1.0373x; 1.0373x over previous
import jax
import jax.numpy as jnp
from jax.experimental import pallas as pl
from jax.experimental.pallas import tpu as pltpu

S = 2048
HID = 128
BSZ = 4
STACK = BSZ * HID
TIME_STEPS = 10
TEMPORAL_STEPS = 3
NSPLIT = 4
RSPLIT = S // NSPLIT
NTRANS = 2 * NSPLIT
NSTAGE = 3


def _swish(v):
    return v * jax.nn.sigmoid(v)


def _dot(a, b, dims):
    return jax.lax.dot_general(
        a, b, (dims, ((), ())), preferred_element_type=jnp.float32)


def _blockmix(w, xs):
    return jnp.concatenate(
        [_dot(w, xs[b * HID:(b + 1) * HID], ((0,), (0,))) for b in range(BSZ)],
        axis=0)


def _col(vec_ref):
    return jnp.transpose(jnp.reshape(vec_ref[...], (1, -1)))


def _col4(vec_ref):
    c = _col(vec_ref)
    return jnp.concatenate([c] * BSZ, axis=0)


def _fused_kernel(x0t_ref, b1_hbm, b2_hbm,
                  w_enc0_ref, b_enc0_ref, w2f_ref, b_enc2_ref,
                  w_c0_ref, w_c2_ref, alpha_ref,
                  w_f_ref, b_f_ref,
                  out_ref, stage, b2bf_s, sems):
    def _issue(i):
        src = b1_hbm if i < NSPLIT else b2_hbm
        c = i % NSPLIT
        pltpu.make_async_copy(src.at[pl.ds(c * RSPLIT, RSPLIT)],
                              stage.at[i % NSTAGE],
                              sems.at[i]).start()

    def _wait(i):
        src = b1_hbm if i < NSPLIT else b2_hbm
        c = i % NSPLIT
        pltpu.make_async_copy(src.at[pl.ds(c * RSPLIT, RSPLIT)],
                              stage.at[i % NSTAGE],
                              sems.at[i]).wait()

    _issue(0)
    _issue(1)

    alpha = alpha_ref[0]
    w_enc0 = w_enc0_ref[...]
    x0h = _swish(jnp.concatenate(
        [_dot(w_enc0, x0t_ref[:, b * S:(b + 1) * S], ((0,), (0,)))
         for b in range(BSZ)], axis=0) + _col4(b_enc0_ref))

    w_f = w_f_ref[...]
    b_f = _col(b_f_ref)
    w_c0 = w_c0_ref[...]

    fillers = (
        [lambda b=b: ('l0', b, _dot(w_c0, x0h[b * HID:(b + 1) * HID],
                                    ((0,), (0,)))) for b in range(BSZ)]
        + [lambda b=b: ('k0', b, _dot(w_f[0:HID], x0h[b * HID:(b + 1) * HID],
                                      ((0,), (0,)))) for b in range(BSZ)]
    )
    l0_1 = [None] * BSZ
    acc = [None] * BSZ

    def _fill(i):
        if i < len(fillers):
            tag, b, v = fillers[i]()
            if tag == 'l0':
                l0_1[b] = v
            else:
                acc[b] = v

    t1 = None
    for i in range(NSPLIT):
        _fill(i)
        _wait(i)
        if i + 2 < NTRANS:
            _issue(i + 2)
        p = _dot(x0h[:, i * RSPLIT:(i + 1) * RSPLIT],
                 stage[i % NSTAGE], ((1,), (0,)))
        t1 = p if t1 is None else t1 + p

    t2 = None
    for i in range(NSPLIT, NTRANS):
        c = i % NSPLIT
        _fill(i)
        _wait(i)
        if i + 2 < NTRANS:
            _issue(i + 2)
        chunk = stage[i % NSTAGE]
        b2bf_s[c * RSPLIT:(c + 1) * RSPLIT, :] = chunk.astype(jnp.bfloat16)
        p = _dot(t1[:, c * RSPLIT:(c + 1) * RSPLIT], chunk, ((1,), (0,)))
        t2 = p if t2 is None else t2 + p
    x2h = _blockmix(w2f_ref[...], t2) + _col4(b_enc2_ref)

    B2 = b2bf_s[...]

    x2h_bf = x2h.astype(jnp.bfloat16)
    up0 = _dot(x2h_bf, B2, ((1,), (0,)))
    l0c = jnp.concatenate(l0_1, axis=0)
    x0h_1 = _swish(up0 + alpha * (l0c - up0))
    x2h_1 = _swish(_blockmix(w_c2_ref[...], x2h))
    acc = [a + _dot(w_f[HID:2 * HID], x0h_1[b * HID:(b + 1) * HID],
                    ((0,), (0,))) for b, a in enumerate(acc)]

    x2h1_bf = x2h_1.astype(jnp.bfloat16)
    l0_2 = _blockmix(w_c0, x0h_1)
    up0b = _dot(x2h1_bf, B2, ((1,), (0,)))
    x0h_2 = _swish(up0b + alpha * (l0_2 - up0b))

    for b in range(BSZ):
        a = acc[b] + _dot(w_f[2 * HID:3 * HID],
                          x0h_2[b * HID:(b + 1) * HID], ((0,), (0,)))
        out_ref[b] = _swish(a + b_f)


def kernel(x, pos, batch, triangles, B1, B2, W_enc0, b_enc0, theta_edge,
           theta_tri, W_enc1, b_enc1, W_enc2, b_enc2, W_conv0, W_conv1,
           W_conv2, alpha, W_tproj, b_tproj, W_dec, b_dec):
    vfull = lambda shp: pl.BlockSpec(shp, lambda: (0,) * len(shp))
    hbm = pl.BlockSpec(memory_space=pl.ANY)
    smem1 = pl.BlockSpec(memory_space=pltpu.SMEM)

    x0t = jnp.concatenate([x.T, pos.T], axis=0)
    hp = 'highest'
    w2f = jnp.dot(jnp.dot(theta_edge, theta_tri, precision=hp), W_enc2,
                  precision=hp)
    w_f = jnp.dot(W_tproj, W_dec, precision=hp)
    b_f = jnp.dot(b_tproj, W_dec, precision=hp) + b_dec

    out = pl.pallas_call(
        _fused_kernel,
        in_specs=[
            vfull((5, BSZ * S)),
            hbm, hbm,
            vfull((5, HID)), vfull((HID,)),
            vfull((HID, HID)), vfull((HID,)),
            vfull((HID, HID)), vfull((HID, HID)),
            smem1,
            vfull((HID * TEMPORAL_STEPS, TIME_STEPS)), vfull((TIME_STEPS,)),
        ],
        out_specs=vfull((BSZ, TIME_STEPS, S)),
        out_shape=jax.ShapeDtypeStruct((BSZ, TIME_STEPS, S), jnp.float32),
        scratch_shapes=[
            pltpu.VMEM((NSTAGE, RSPLIT, S), jnp.float32),
            pltpu.VMEM((S, S), jnp.bfloat16),
            pltpu.SemaphoreType.DMA((NTRANS,)),
        ],
        compiler_params=pltpu.CompilerParams(
            vmem_limit_bytes=110 * 1024 * 1024),
    )(
        x0t, B1, B2,
        W_enc0, b_enc0, w2f, b_enc2,
        W_conv0, W_conv2, alpha.reshape(1),
        w_f, b_f,
    )
    return out
